# Initial kernel scaffold; baseline (speedup 1.0000x reference)
import jax
import jax.numpy as jnp
from jax.experimental import pallas as pl


def kernel(x, target):
    raise NotImplementedError("write your pallas kernel here")


# speedup vs baseline: 3.0258x; 3.0258x over previous
import functools
import math

import jax
import jax.numpy as jnp
from jax import lax
from jax.experimental import pallas as pl
from jax.experimental.pallas import tpu as pltpu
from jax.experimental.pallas import tpu_sc as plsc

N = 4096
SIZE = 16384
PADDING_IDX = 0
SMOOTHING = 0.1
CONFIDENCE = 1.0 - SMOOTHING
S = SMOOTHING / (SIZE - 2)
C_CONST = (SIZE - 2) * S * math.log(S) + CONFIDENCE * math.log(CONFIDENCE)

NUM_CORES = 2
NUM_SUBCORES = 16
LANES = 16
NUM_WORKERS = NUM_CORES * NUM_SUBCORES
ROWS_PER_WORKER = N // NUM_WORKERS

BLOCK_ROWS = 64


def _sc_coeffs_body(x_flat_hbm, tgt_hbm, a_hbm, b_hbm,
                    tgt_v, idx_v, g_v, a_v, b_v, sem):
    wid = lax.axis_index("s") * NUM_CORES + lax.axis_index("c")
    base = wid * ROWS_PER_WORKER
    pltpu.sync_copy(tgt_hbm.at[pl.ds(base, ROWS_PER_WORKER)], tgt_v)
    for k in range(ROWS_PER_WORKER // LANES):
        t = tgt_v[pl.ds(k * LANES, LANES)]
        rows = lax.broadcasted_iota(jnp.int32, (LANES,), 0) + (base + k * LANES)
        idx_v[pl.ds(k * LANES, LANES)] = rows * SIZE + t
    pltpu.async_copy(x_flat_hbm.at[idx_v], g_v, sem).wait()
    for k in range(ROWS_PER_WORKER // LANES):
        sl = pl.ds(k * LANES, LANES)
        t = tgt_v[sl]
        g = g_v[sl]
        mask = t != PADDING_IDX
        a_v[sl] = jnp.where(mask, C_CONST + (S - CONFIDENCE) * g, 0.0)
        b_v[sl] = jnp.where(mask, -S, 0.0)
    pltpu.sync_copy(a_v, a_hbm.at[pl.ds(base, ROWS_PER_WORKER)])
    pltpu.sync_copy(b_v, b_hbm.at[pl.ds(base, ROWS_PER_WORKER)])


def _sc_coeffs(x_flat, target_i32):
    mesh = plsc.VectorSubcoreMesh(core_axis_name="c", subcore_axis_name="s")
    f = functools.partial(
        pl.kernel,
        mesh=mesh,
        out_type=[
            jax.ShapeDtypeStruct((N,), jnp.float32),
            jax.ShapeDtypeStruct((N,), jnp.float32),
        ],
        scratch_types=[
            pltpu.VMEM((ROWS_PER_WORKER,), jnp.int32),
            pltpu.VMEM((ROWS_PER_WORKER,), jnp.int32),
            pltpu.VMEM((ROWS_PER_WORKER,), jnp.float32),
            pltpu.VMEM((ROWS_PER_WORKER,), jnp.float32),
            pltpu.VMEM((ROWS_PER_WORKER,), jnp.float32),
            pltpu.SemaphoreType.DMA,
        ],
    )(_sc_coeffs_body)
    return f(x_flat, target_i32)


def _tc_body(a_ref, b_ref, x_ref, out_ref):
    i = pl.program_id(0)
    block = x_ref[...]
    a = a_ref[0]
    b = b_ref[0]
    x0 = block[:, 0:1]
    partial = jnp.sum(a) + jnp.sum(block * b) - jnp.sum(b * x0)

    @pl.when(i == 0)
    def _init():
        out_ref[0, 0] = partial

    @pl.when(i > 0)
    def _acc():
        out_ref[0, 0] += partial


def _tc_reduce(x, a, b):
    num_blocks = N // BLOCK_ROWS
    a3 = a.reshape(num_blocks, BLOCK_ROWS, 1)
    b3 = b.reshape(num_blocks, BLOCK_ROWS, 1)
    out = pl.pallas_call(
        _tc_body,
        grid=(num_blocks,),
        in_specs=[
            pl.BlockSpec((1, BLOCK_ROWS, 1), lambda i: (i, 0, 0)),
            pl.BlockSpec((1, BLOCK_ROWS, 1), lambda i: (i, 0, 0)),
            pl.BlockSpec((BLOCK_ROWS, SIZE), lambda i: (i, 0)),
        ],
        out_specs=pl.BlockSpec(
            (1, 1), lambda i: (0, 0), memory_space=pltpu.SMEM),
        out_shape=jax.ShapeDtypeStruct((1, 1), jnp.float32),
    )(a3, b3, x)
    return out[0, 0]


def kernel(x, target):
    target_i32 = target.astype(jnp.int32)
    x_flat = x.reshape(N * SIZE)
    a, b = _sc_coeffs(x_flat, target_i32)
    return _tc_reduce(x, a, b)


# speedup vs baseline: 7.8348x; 2.5893x over previous
import math

import jax
import jax.numpy as jnp
from jax import lax
from jax.experimental import pallas as pl
from jax.experimental.pallas import tpu as pltpu
from jax.experimental.pallas import tpu_sc as plsc

N = 4096
SIZE = 16384
PADDING_IDX = 0
SMOOTHING = 0.1
CONFIDENCE = 1.0 - SMOOTHING
S = SMOOTHING / (SIZE - 2)
C_CONST = (SIZE - 2) * S * math.log(S) + CONFIDENCE * math.log(CONFIDENCE)

LANES = 16

BLOCK_ROWS = 64
NUM_BLOCKS = N // BLOCK_ROWS


def _tc_body(t_ref, x_ref, p_ref):
    block = x_ref[...]
    t = t_ref[0]
    col = lax.broadcasted_iota(jnp.int32, (BLOCK_ROWS, SIZE), 1)
    g = jnp.sum(jnp.where(col == t, block, 0.0), axis=1, keepdims=True)
    rowsum = jnp.sum(block, axis=1, keepdims=True)
    q = rowsum - block[:, 0:1]
    p_ref[0] = C_CONST - S * q + (S - CONFIDENCE) * g


def _tc_partials(x, target_i32):
    t3 = target_i32.reshape(NUM_BLOCKS, BLOCK_ROWS, 1)
    return pl.pallas_call(
        _tc_body,
        grid=(NUM_BLOCKS,),
        in_specs=[
            pl.BlockSpec((1, BLOCK_ROWS, 1), lambda i: (i, 0, 0)),
            pl.BlockSpec((BLOCK_ROWS, SIZE), lambda i: (i, 0)),
        ],
        out_specs=pl.BlockSpec((1, BLOCK_ROWS, 1), lambda i: (i, 0, 0)),
        out_shape=jax.ShapeDtypeStruct((NUM_BLOCKS, BLOCK_ROWS, 1), jnp.float32),
    )(t3, x)


def _sc_body(t_hbm, p_hbm, out_hbm, t_v, p_v, o_v):
    c = lax.axis_index("c")
    s = lax.axis_index("s")

    @pl.when(jnp.logical_and(c == 0, s == 0))
    def _():
        pltpu.sync_copy(t_hbm, t_v)
        pltpu.sync_copy(p_hbm, p_v)
        acc = jnp.zeros((LANES,), jnp.float32)
        for k in range(N // LANES):
            sl = pl.ds(k * LANES, LANES)
            acc = acc + jnp.where(t_v[sl] != PADDING_IDX, p_v[sl], 0.0)
        lane = lax.broadcasted_iota(jnp.int32, (LANES,), 0)
        dnums = lax.GatherDimensionNumbers(
            offset_dims=(), collapsed_slice_dims=(0,), start_index_map=(0,))
        for sh in (8, 4, 2, 1):
            idx = jnp.bitwise_and(lane + sh, LANES - 1)
            acc = acc + lax.gather(
                acc, idx[:, None], dimension_numbers=dnums, slice_sizes=(1,),
                mode=lax.GatherScatterMode.PROMISE_IN_BOUNDS)
        o_v[...] = acc
        pltpu.sync_copy(o_v, out_hbm)


def _sc_masked_sum(target_i32, p_flat):
    mesh = plsc.VectorSubcoreMesh(core_axis_name="c", subcore_axis_name="s")
    f = pl.kernel(
        _sc_body,
        mesh=mesh,
        out_type=jax.ShapeDtypeStruct((LANES,), jnp.float32),
        scratch_types=[
            pltpu.VMEM((N,), jnp.int32),
            pltpu.VMEM((N,), jnp.float32),
            pltpu.VMEM((LANES,), jnp.float32),
        ],
    )
    return f(target_i32, p_flat)


def kernel(x, target):
    target_i32 = target.astype(jnp.int32)
    p = _tc_partials(x, target_i32).reshape(N)
    return _sc_masked_sum(target_i32, p)[0]


# speedup vs baseline: 9.4229x; 1.2027x over previous
import math

import jax
import jax.numpy as jnp
from jax import lax
from jax.experimental import pallas as pl
from jax.experimental.pallas import tpu as pltpu
from jax.experimental.pallas import tpu_sc as plsc

N = 4096
SIZE = 16384
PADDING_IDX = 0
SMOOTHING = 0.1
CONFIDENCE = 1.0 - SMOOTHING
S = SMOOTHING / (SIZE - 2)
C_CONST = (SIZE - 2) * S * math.log(S) + CONFIDENCE * math.log(CONFIDENCE)

LANES = 16

BLOCK_ROWS = 128
NUM_BLOCKS = N // BLOCK_ROWS


def _tc_body(t_ref, x_ref, p_ref):
    block = x_ref[...]
    t = t_ref[0]
    col = lax.broadcasted_iota(jnp.int32, (BLOCK_ROWS, SIZE), 1)
    g = jnp.sum(jnp.where(col == t, block, 0.0), axis=1, keepdims=True)
    rowsum = jnp.sum(block, axis=1, keepdims=True)
    q = rowsum - block[:, 0:1]
    p_ref[0] = C_CONST - S * q + (S - CONFIDENCE) * g


def _tc_partials(x, target_i32):
    t3 = target_i32.reshape(NUM_BLOCKS, BLOCK_ROWS, 1)
    return pl.pallas_call(
        _tc_body,
        grid=(NUM_BLOCKS,),
        in_specs=[
            pl.BlockSpec((1, BLOCK_ROWS, 1), lambda i: (i, 0, 0)),
            pl.BlockSpec((BLOCK_ROWS, SIZE), lambda i: (i, 0)),
        ],
        out_specs=pl.BlockSpec((1, BLOCK_ROWS, 1), lambda i: (i, 0, 0)),
        out_shape=jax.ShapeDtypeStruct((NUM_BLOCKS, BLOCK_ROWS, 1), jnp.float32),
    )(t3, x)


def _sc_body(t_hbm, p_hbm, out_hbm, t_v, p_v, o_v):
    c = lax.axis_index("c")
    s = lax.axis_index("s")

    @pl.when(jnp.logical_and(c == 0, s == 0))
    def _():
        pltpu.sync_copy(t_hbm, t_v)
        pltpu.sync_copy(p_hbm, p_v)
        acc = jnp.zeros((LANES,), jnp.float32)
        for k in range(N // LANES):
            sl = pl.ds(k * LANES, LANES)
            acc = acc + jnp.where(t_v[sl] != PADDING_IDX, p_v[sl], 0.0)
        lane = lax.broadcasted_iota(jnp.int32, (LANES,), 0)
        dnums = lax.GatherDimensionNumbers(
            offset_dims=(), collapsed_slice_dims=(0,), start_index_map=(0,))
        for sh in (8, 4, 2, 1):
            idx = jnp.bitwise_and(lane + sh, LANES - 1)
            acc = acc + lax.gather(
                acc, idx[:, None], dimension_numbers=dnums, slice_sizes=(1,),
                mode=lax.GatherScatterMode.PROMISE_IN_BOUNDS)
        o_v[...] = acc
        pltpu.sync_copy(o_v, out_hbm)


def _sc_masked_sum(target_i32, p_flat):
    mesh = plsc.VectorSubcoreMesh(core_axis_name="c", subcore_axis_name="s")
    f = pl.kernel(
        _sc_body,
        mesh=mesh,
        out_type=jax.ShapeDtypeStruct((LANES,), jnp.float32),
        scratch_types=[
            pltpu.VMEM((N,), jnp.int32),
            pltpu.VMEM((N,), jnp.float32),
            pltpu.VMEM((LANES,), jnp.float32),
        ],
    )
    return f(target_i32, p_flat)


def kernel(x, target):
    target_i32 = target.astype(jnp.int32)
    p = _tc_partials(x, target_i32).reshape(N)
    return _sc_masked_sum(target_i32, p)[0]
